# baseline (speedup 1.0000x reference)
import functools

import jax
import jax.numpy as jnp
from jax import lax
from jax.experimental import pallas as pl
from jax.experimental.pallas import tpu as pltpu
from jax.experimental.pallas import tpu_sc as plsc

N_NODES = 10000
N_EDGES = 320000
HIDDEN = 128
F_LAST = 16

NC = 2
NS = 16
LANES = 16

CHUNK = 128
N_CHUNKS = 2560
PAD_EDGES = N_CHUNKS * CHUNK
TILE_CHUNKS = N_CHUNKS // (NC * NS)
HIST_ROWS = N_NODES + LANES

HALF = 5120
ACC_ROWS = HALF + LANES
TAIL = N_NODES - HALF


@functools.cache
def _mesh():
    return plsc.VectorSubcoreMesh(
        core_axis_name="c", subcore_axis_name="s", num_cores=NC, num_subcores=NS
    )


@functools.cache
def _make_deg():
    @functools.partial(
        pl.kernel,
        out_type=jax.ShapeDtypeStruct((NC, 1, N_NODES), jnp.float32),
        mesh=_mesh(),
        scratch_types=[
            pltpu.VMEM((TILE_CHUNKS, CHUNK), jnp.int32),
            pltpu.VMEM((CHUNK,), jnp.float32),
            pltpu.VMEM((10240,), jnp.float32),
            pltpu.VMEM_SHARED((HIST_ROWS,), jnp.float32),
        ],
    )
    def _deg_kernel(col2_hbm, deg_out, col_all, ones_v, zbuf, deg_acc):
        c = lax.axis_index("c")
        s = lax.axis_index("s")

        def _fill_ones(i, carry):
            ones_v[pl.ds(i * 16, 16)] = jnp.ones((16,), jnp.float32)
            return carry

        lax.fori_loop(0, CHUNK // 16, _fill_ones, 0)

        def _fill_zero(i, carry):
            zbuf[pl.ds(i * 16, 16)] = jnp.zeros((16,), jnp.float32)
            return carry

        lax.fori_loop(0, 10240 // 16, _fill_zero, 0)

        @pl.when(s == 0)
        def _():
            pltpu.sync_copy(zbuf.at[pl.ds(0, HIST_ROWS)], deg_acc)

        plsc.subcore_barrier()

        base = (c * NS + s) * TILE_CHUNKS
        pltpu.sync_copy(col2_hbm.at[pl.ds(base, TILE_CHUNKS)], col_all)

        def _step(k, carry):
            pltpu.sync_copy(ones_v, deg_acc.at[col_all.at[k]], add=True)
            return carry

        lax.fori_loop(0, TILE_CHUNKS, _step, 0)
        plsc.subcore_barrier()

        @pl.when(s == 0)
        def _():
            pltpu.sync_copy(deg_acc.at[pl.ds(0, N_NODES)], zbuf.at[pl.ds(0, N_NODES)])
            pltpu.sync_copy(zbuf.at[pl.ds(0, N_NODES)], deg_out.at[c, 0])

    return _deg_kernel


@functools.cache
def _make_scatter_half():
    @functools.partial(
        pl.kernel,
        out_type=jax.ShapeDtypeStruct((NC, HALF, HIDDEN), jnp.float32),
        mesh=_mesh(),
        scratch_types=[
            pltpu.VMEM((TILE_CHUNKS + 1, CHUNK), jnp.int32),
            pltpu.VMEM((TILE_CHUNKS, CHUNK), jnp.int32),
            pltpu.VMEM((CHUNK, HIDDEN), jnp.float32),
            pltpu.VMEM((CHUNK, HIDDEN), jnp.float32),
            pltpu.VMEM((128, HIDDEN), jnp.float32),
            pltpu.VMEM_SHARED((ACC_ROWS, HIDDEN), jnp.float32),
            pltpu.SemaphoreType.DMA,
            pltpu.SemaphoreType.DMA,
        ],
    )
    def _scatter(y_hbm, row2_hbm, colh_hbm, z_out,
                 row_all, col_all, rows_a, rows_b, zbuf, z_acc, sema, semb):
        c = lax.axis_index("c")
        s = lax.axis_index("s")

        def _zero(i, carry):
            for j in range(HIDDEN // 16):
                zbuf[i, pl.ds(j * 16, 16)] = jnp.zeros((16,), jnp.float32)
            return carry

        lax.fori_loop(0, 128, _zero, 0)

        for j in range(CHUNK // 16):
            row_all[TILE_CHUNKS, pl.ds(j * 16, 16)] = jnp.zeros((16,), jnp.int32)

        r0 = s * (ACC_ROWS // NS)
        for off, n in ((0, 128), (128, 128), (256, 65)):
            pltpu.sync_copy(zbuf.at[pl.ds(0, n)], z_acc.at[pl.ds(r0 + off, n)])
        plsc.subcore_barrier()

        base = (c * NS + s) * TILE_CHUNKS
        pltpu.sync_copy(row2_hbm.at[pl.ds(base, TILE_CHUNKS)],
                        row_all.at[pl.ds(0, TILE_CHUNKS)])
        pltpu.sync_copy(colh_hbm.at[pl.ds(base, TILE_CHUNKS)], col_all)

        pltpu.async_copy(y_hbm.at[row_all.at[0]], rows_a, sema)

        def _pair(j, carry):
            k = 2 * j
            pltpu.async_copy(y_hbm.at[row_all.at[k + 1]], rows_b, semb)
            pltpu.make_async_copy(y_hbm.at[row_all.at[k]], rows_a, sema).wait()
            pltpu.sync_copy(rows_a, z_acc.at[col_all.at[k]], add=True)
            pltpu.async_copy(y_hbm.at[row_all.at[k + 2]], rows_a, sema)
            pltpu.make_async_copy(y_hbm.at[row_all.at[k + 1]], rows_b, semb).wait()
            pltpu.sync_copy(rows_b, z_acc.at[col_all.at[k + 1]], add=True)
            return carry

        lax.fori_loop(0, TILE_CHUNKS // 2, _pair, 0)
        pltpu.make_async_copy(y_hbm.at[row_all.at[0]], rows_a, sema).wait()
        plsc.subcore_barrier()

        d0 = s * (HALF // NS)
        pltpu.sync_copy(z_acc.at[pl.ds(d0, HALF // NS)], z_out.at[c, pl.ds(d0, HALF // NS)])

    return _scatter


def _dinv_body(deg_ref, dinv_ref):
    dinv_ref[...] = lax.rsqrt(deg_ref[0] + deg_ref[1] + 1.0)


def _dinv_call(deg_pair):
    return pl.pallas_call(
        _dinv_body,
        out_shape=jax.ShapeDtypeStruct((N_NODES, 1), jnp.float32),
    )(deg_pair.reshape(NC, N_NODES, 1))


def _mm1_body(x_ref, w_ref, dinv_ref, y_ref):
    y_ref[...] = (
        jnp.dot(x_ref[...], w_ref[...], preferred_element_type=jnp.float32)
        * dinv_ref[...]
    )


def _mm1_call(x, w, dinv):
    bm = 400
    kdim = x.shape[1]
    return pl.pallas_call(
        _mm1_body,
        grid=(N_NODES // bm,),
        in_specs=[
            pl.BlockSpec((bm, kdim), lambda i: (i, 0)),
            pl.BlockSpec((kdim, HIDDEN), lambda i: (0, 0)),
            pl.BlockSpec((bm, 1), lambda i: (i, 0)),
        ],
        out_specs=pl.BlockSpec((bm, HIDDEN), lambda i: (i, 0)),
        out_shape=jax.ShapeDtypeStruct((N_NODES, HIDDEN), jnp.float32),
    )(x, w, dinv)


def _mid_body(z_ref, y_ref, dinv_ref, b_ref, w_ref, o_ref):
    t = (z_ref[0] + z_ref[1] + y_ref[...]) * dinv_ref[...] + b_ref[...]
    t = jnp.maximum(t, 0.0)
    o_ref[...] = (
        jnp.dot(t, w_ref[...], preferred_element_type=jnp.float32) * dinv_ref[...]
    )


def _mid_call(z_pair, y, dinv, b, w):
    bm = 1000
    return pl.pallas_call(
        _mid_body,
        grid=(N_NODES // bm,),
        in_specs=[
            pl.BlockSpec((NC, bm, HIDDEN), lambda i: (0, i, 0)),
            pl.BlockSpec((bm, HIDDEN), lambda i: (i, 0)),
            pl.BlockSpec((bm, 1), lambda i: (i, 0)),
            pl.BlockSpec((1, HIDDEN), lambda i: (0, 0)),
            pl.BlockSpec((HIDDEN, HIDDEN), lambda i: (0, 0)),
        ],
        out_specs=pl.BlockSpec((bm, HIDDEN), lambda i: (i, 0)),
        out_shape=jax.ShapeDtypeStruct((N_NODES, HIDDEN), jnp.float32),
    )(z_pair, y, dinv, b.reshape(1, HIDDEN), w)


def _pre10_body(z_ref, y_ref, dinv_ref, b_ref, o_ref):
    t = (z_ref[0] + z_ref[1] + y_ref[...]) * dinv_ref[...] + b_ref[...]
    o_ref[...] = jnp.maximum(t, 0.0) * dinv_ref[...]


def _pre10_call(z_pair, y, dinv, b):
    bm = 1000
    return pl.pallas_call(
        _pre10_body,
        grid=(N_NODES // bm,),
        in_specs=[
            pl.BlockSpec((NC, bm, HIDDEN), lambda i: (0, i, 0)),
            pl.BlockSpec((bm, HIDDEN), lambda i: (i, 0)),
            pl.BlockSpec((bm, 1), lambda i: (i, 0)),
            pl.BlockSpec((1, HIDDEN), lambda i: (0, 0)),
        ],
        out_specs=pl.BlockSpec((bm, HIDDEN), lambda i: (i, 0)),
        out_shape=jax.ShapeDtypeStruct((N_NODES, HIDDEN), jnp.float32),
    )(z_pair, y, dinv, b.reshape(1, HIDDEN))


def _fin_body(z_ref, u_ref, dinv_ref, w_ref, b_ref, o_ref):
    t = (z_ref[0] + z_ref[1] + u_ref[...]) * dinv_ref[...]
    o_ref[...] = (
        jnp.dot(t, w_ref[...], preferred_element_type=jnp.float32) + b_ref[...]
    )


def _fin_call(z_pair, u, dinv, w, b):
    bm = 1000
    return pl.pallas_call(
        _fin_body,
        grid=(N_NODES // bm,),
        in_specs=[
            pl.BlockSpec((NC, bm, HIDDEN), lambda i: (0, i, 0)),
            pl.BlockSpec((bm, HIDDEN), lambda i: (i, 0)),
            pl.BlockSpec((bm, 1), lambda i: (i, 0)),
            pl.BlockSpec((HIDDEN, F_LAST), lambda i: (0, 0)),
            pl.BlockSpec((1, F_LAST), lambda i: (0, 0)),
        ],
        out_specs=pl.BlockSpec((bm, F_LAST), lambda i: (i, 0)),
        out_shape=jax.ShapeDtypeStruct((N_NODES, F_LAST), jnp.float32),
    )(z_pair, u, dinv, w, b.reshape(1, F_LAST))


def kernel(x, edge_index, W1, b1, W2, b2, W3, b3, W4, b4, W5, b5, W6, b6,
           W7, b7, W8, b8, W9, b9, W10, b10):
    ei = edge_index.astype(jnp.int32)
    row, col = ei[0], ei[1]
    pad = PAD_EDGES - N_EDGES
    spread = jnp.arange(pad, dtype=jnp.int32) % LANES
    row_p = jnp.concatenate(
        [row, jnp.zeros((pad,), jnp.int32)]).reshape(N_CHUNKS, CHUNK)
    col_p = jnp.concatenate(
        [col, N_NODES + spread]).reshape(N_CHUNKS, CHUNK)

    espread = jnp.arange(N_EDGES, dtype=jnp.int32) % LANES
    dummy = HALF + espread
    col_h0 = jnp.concatenate(
        [jnp.where(col < HALF, col, dummy), HALF + spread]
    ).reshape(N_CHUNKS, CHUNK)
    col_h1 = jnp.concatenate(
        [jnp.where(col >= HALF, col - HALF, dummy), HALF + spread]
    ).reshape(N_CHUNKS, CHUNK)

    _deg = _make_deg()
    _sh = _make_scatter_half()

    deg_pair = _deg(col_p).reshape(NC, N_NODES)
    dinv = _dinv_call(deg_pair)

    def edge_phase(y):
        z0 = _sh(y, row_p, col_h0)
        z1 = _sh(y, row_p, col_h1)
        return jnp.concatenate([z0, z1[:, :TAIL]], axis=1)

    Ws = [W1, W2, W3, W4, W5, W6, W7, W8, W9]
    bs = [b1, b2, b3, b4, b5, b6, b7, b8, b9]
    w10p = jnp.pad(W10, ((0, 0), (0, F_LAST - W10.shape[1])))
    b10p = jnp.pad(b10, (0, F_LAST - b10.shape[0]))

    y = _mm1_call(x, Ws[0], dinv)
    for i in range(1, 9):
        z = edge_phase(y)
        y = _mid_call(z, y, dinv, bs[i - 1], Ws[i])
    z = edge_phase(y)
    u = _pre10_call(z, y, dinv, bs[8])
    z = edge_phase(u)
    out16 = _fin_call(z, u, dinv, w10p, b10p)
    return out16[:, : b10.shape[0]]


# speedup vs baseline: 1.8086x; 1.8086x over previous
import functools

import jax
import jax.numpy as jnp
from jax import lax
from jax.experimental import pallas as pl
from jax.experimental.pallas import tpu as pltpu
from jax.experimental.pallas import tpu_sc as plsc

N_NODES = 10000
N_EDGES = 320000
HIDDEN = 128
F_LAST = 16

NC = 2
NS = 16
LANES = 16

CHUNK = 128
N_CHUNKS = 2560
PAD_EDGES = N_CHUNKS * CHUNK
TILE_CHUNKS = N_CHUNKS // (NC * NS)
HIST_ROWS = N_NODES + LANES

HALF = 5120
ACC_ROWS = HALF + LANES
TAIL = N_NODES - HALF


@functools.cache
def _mesh():
    return plsc.VectorSubcoreMesh(
        core_axis_name="c", subcore_axis_name="s", num_cores=NC, num_subcores=NS
    )


@functools.cache
def _make_deg():
    @functools.partial(
        pl.kernel,
        out_type=jax.ShapeDtypeStruct((NC, 1, N_NODES), jnp.float32),
        mesh=_mesh(),
        scratch_types=[
            pltpu.VMEM((TILE_CHUNKS, CHUNK), jnp.int32),
            pltpu.VMEM((CHUNK,), jnp.float32),
            pltpu.VMEM((10240,), jnp.float32),
            pltpu.VMEM_SHARED((HIST_ROWS,), jnp.float32),
        ],
    )
    def _deg_kernel(col2_hbm, deg_out, col_all, ones_v, zbuf, deg_acc):
        c = lax.axis_index("c")
        s = lax.axis_index("s")

        def _fill_ones(i, carry):
            ones_v[pl.ds(i * 16, 16)] = jnp.ones((16,), jnp.float32)
            return carry

        lax.fori_loop(0, CHUNK // 16, _fill_ones, 0)

        def _fill_zero(i, carry):
            zbuf[pl.ds(i * 16, 16)] = jnp.zeros((16,), jnp.float32)
            return carry

        lax.fori_loop(0, 10240 // 16, _fill_zero, 0)

        @pl.when(s == 0)
        def _():
            pltpu.sync_copy(zbuf.at[pl.ds(0, HIST_ROWS)], deg_acc)

        plsc.subcore_barrier()

        base = (c * NS + s) * TILE_CHUNKS
        pltpu.sync_copy(col2_hbm.at[pl.ds(base, TILE_CHUNKS)], col_all)

        def _step(k, carry):
            pltpu.sync_copy(ones_v, deg_acc.at[col_all.at[k]], add=True)
            return carry

        lax.fori_loop(0, TILE_CHUNKS, _step, 0)
        plsc.subcore_barrier()

        @pl.when(s == 0)
        def _():
            pltpu.sync_copy(deg_acc.at[pl.ds(0, N_NODES)], zbuf.at[pl.ds(0, N_NODES)])
            pltpu.sync_copy(zbuf.at[pl.ds(0, N_NODES)], deg_out.at[c, 0])

    return _deg_kernel


@functools.cache
def _make_scatter_half():
    @functools.partial(
        pl.kernel,
        out_type=jax.ShapeDtypeStruct((NC, HALF, HIDDEN), jnp.float32),
        mesh=_mesh(),
        scratch_types=[
            pltpu.VMEM((TILE_CHUNKS, CHUNK), jnp.int32),
            pltpu.VMEM((TILE_CHUNKS, CHUNK), jnp.int32),
            pltpu.VMEM((CHUNK, HIDDEN), jnp.float32),
            pltpu.VMEM((128, HIDDEN), jnp.float32),
            pltpu.VMEM_SHARED((ACC_ROWS, HIDDEN), jnp.float32),
            pltpu.VMEM((LANES,), jnp.int32),
            pltpu.SemaphoreType.DMA,
        ],
    )
    def _scatter(y_hbm, row2_hbm, colh_hbm, cnt_hbm, z_out,
                 row_all, col_all, rows_v, zbuf, z_acc, cnt_v, sem):
        c = lax.axis_index("c")
        s = lax.axis_index("s")

        def _zero(i, carry):
            for j in range(HIDDEN // 16):
                zbuf[i, pl.ds(j * 16, 16)] = jnp.zeros((16,), jnp.float32)
            return carry

        lax.fori_loop(0, 128, _zero, 0)

        r0 = s * (ACC_ROWS // NS)
        for off, n in ((0, 128), (128, 128), (256, 65)):
            pltpu.sync_copy(zbuf.at[pl.ds(0, n)], z_acc.at[pl.ds(r0 + off, n)])
        plsc.subcore_barrier()

        wid = c * NS + s
        pltpu.sync_copy(cnt_hbm.at[wid], cnt_v)
        cnt = cnt_v[...][0]

        base = wid * TILE_CHUNKS
        pltpu.sync_copy(row2_hbm.at[pl.ds(base, TILE_CHUNKS)], row_all)
        pltpu.sync_copy(colh_hbm.at[pl.ds(base, TILE_CHUNKS)], col_all)

        def _step(k, carry):
            pltpu.async_copy(y_hbm.at[row_all.at[k]], rows_v, sem).wait()
            pltpu.sync_copy(rows_v, z_acc.at[col_all.at[k]], add=True)
            return carry

        lax.fori_loop(0, cnt, _step, 0)
        plsc.subcore_barrier()

        d0 = s * (HALF // NS)
        pltpu.sync_copy(z_acc.at[pl.ds(d0, HALF // NS)], z_out.at[c, pl.ds(d0, HALF // NS)])

    return _scatter


def _dinv_body(deg_ref, dinv_ref):
    dinv_ref[...] = lax.rsqrt(deg_ref[0] + deg_ref[1] + 1.0)


def _dinv_call(deg_pair):
    return pl.pallas_call(
        _dinv_body,
        out_shape=jax.ShapeDtypeStruct((N_NODES, 1), jnp.float32),
    )(deg_pair.reshape(NC, N_NODES, 1))


def _mm1_body(x_ref, w_ref, dinv_ref, y_ref):
    y_ref[...] = (
        jnp.dot(x_ref[...], w_ref[...], preferred_element_type=jnp.float32)
        * dinv_ref[...]
    )


def _mm1_call(x, w, dinv):
    bm = 400
    kdim = x.shape[1]
    return pl.pallas_call(
        _mm1_body,
        grid=(N_NODES // bm,),
        in_specs=[
            pl.BlockSpec((bm, kdim), lambda i: (i, 0)),
            pl.BlockSpec((kdim, HIDDEN), lambda i: (0, 0)),
            pl.BlockSpec((bm, 1), lambda i: (i, 0)),
        ],
        out_specs=pl.BlockSpec((bm, HIDDEN), lambda i: (i, 0)),
        out_shape=jax.ShapeDtypeStruct((N_NODES, HIDDEN), jnp.float32),
    )(x, w, dinv)


def _mid_body(z_ref, y_ref, dinv_ref, b_ref, w_ref, o_ref):
    t = (z_ref[0] + z_ref[1] + y_ref[...]) * dinv_ref[...] + b_ref[...]
    t = jnp.maximum(t, 0.0)
    o_ref[...] = (
        jnp.dot(t, w_ref[...], preferred_element_type=jnp.float32) * dinv_ref[...]
    )


def _mid_call(z_pair, y, dinv, b, w):
    bm = 1000
    return pl.pallas_call(
        _mid_body,
        grid=(N_NODES // bm,),
        in_specs=[
            pl.BlockSpec((NC, bm, HIDDEN), lambda i: (0, i, 0)),
            pl.BlockSpec((bm, HIDDEN), lambda i: (i, 0)),
            pl.BlockSpec((bm, 1), lambda i: (i, 0)),
            pl.BlockSpec((1, HIDDEN), lambda i: (0, 0)),
            pl.BlockSpec((HIDDEN, HIDDEN), lambda i: (0, 0)),
        ],
        out_specs=pl.BlockSpec((bm, HIDDEN), lambda i: (i, 0)),
        out_shape=jax.ShapeDtypeStruct((N_NODES, HIDDEN), jnp.float32),
    )(z_pair, y, dinv, b.reshape(1, HIDDEN), w)


def _pre10_body(z_ref, y_ref, dinv_ref, b_ref, o_ref):
    t = (z_ref[0] + z_ref[1] + y_ref[...]) * dinv_ref[...] + b_ref[...]
    o_ref[...] = jnp.maximum(t, 0.0) * dinv_ref[...]


def _pre10_call(z_pair, y, dinv, b):
    bm = 1000
    return pl.pallas_call(
        _pre10_body,
        grid=(N_NODES // bm,),
        in_specs=[
            pl.BlockSpec((NC, bm, HIDDEN), lambda i: (0, i, 0)),
            pl.BlockSpec((bm, HIDDEN), lambda i: (i, 0)),
            pl.BlockSpec((bm, 1), lambda i: (i, 0)),
            pl.BlockSpec((1, HIDDEN), lambda i: (0, 0)),
        ],
        out_specs=pl.BlockSpec((bm, HIDDEN), lambda i: (i, 0)),
        out_shape=jax.ShapeDtypeStruct((N_NODES, HIDDEN), jnp.float32),
    )(z_pair, y, dinv, b.reshape(1, HIDDEN))


def _fin_body(z_ref, u_ref, dinv_ref, w_ref, b_ref, o_ref):
    t = (z_ref[0] + z_ref[1] + u_ref[...]) * dinv_ref[...]
    o_ref[...] = (
        jnp.dot(t, w_ref[...], preferred_element_type=jnp.float32) + b_ref[...]
    )


def _fin_call(z_pair, u, dinv, w, b):
    bm = 1000
    return pl.pallas_call(
        _fin_body,
        grid=(N_NODES // bm,),
        in_specs=[
            pl.BlockSpec((NC, bm, HIDDEN), lambda i: (0, i, 0)),
            pl.BlockSpec((bm, HIDDEN), lambda i: (i, 0)),
            pl.BlockSpec((bm, 1), lambda i: (i, 0)),
            pl.BlockSpec((HIDDEN, F_LAST), lambda i: (0, 0)),
            pl.BlockSpec((1, F_LAST), lambda i: (0, 0)),
        ],
        out_specs=pl.BlockSpec((bm, F_LAST), lambda i: (i, 0)),
        out_shape=jax.ShapeDtypeStruct((N_NODES, F_LAST), jnp.float32),
    )(z_pair, u, dinv, w, b.reshape(1, F_LAST))


def kernel(x, edge_index, W1, b1, W2, b2, W3, b3, W4, b4, W5, b5, W6, b6,
           W7, b7, W8, b8, W9, b9, W10, b10):
    ei = edge_index.astype(jnp.int32)
    row, col = ei[0], ei[1]
    pad = PAD_EDGES - N_EDGES
    spread = jnp.arange(pad, dtype=jnp.int32) % LANES
    row_p = jnp.concatenate(
        [row, jnp.zeros((pad,), jnp.int32)]).reshape(N_CHUNKS, CHUNK)
    col_p = jnp.concatenate(
        [col, N_NODES + spread]).reshape(N_CHUNKS, CHUNK)

    NW = NC * NS
    widx = jnp.arange(NW, dtype=jnp.int32)

    def _partition(base):
        mask = (col >= base) & (col < base + HALF)
        pos = jnp.cumsum(mask) - 1
        ch = pos // CHUNK
        addr = (ch % NW) * (TILE_CHUNKS * CHUNK) + (ch // NW) * CHUNK + pos % CHUNK
        addr = jnp.where(mask, addr, PAD_EDGES)
        rbuf = jnp.zeros((PAD_EDGES + 1,), jnp.int32)
        cinit = HALF + jnp.arange(PAD_EDGES + 1, dtype=jnp.int32) % LANES
        rbuf = rbuf.at[addr].set(row)[:PAD_EDGES].reshape(N_CHUNKS, CHUNK)
        cbuf = cinit.at[addr].set(col - base)[:PAD_EDGES].reshape(N_CHUNKS, CHUNK)
        nch = (jnp.sum(mask.astype(jnp.int32)) + CHUNK - 1) // CHUNK
        cnt = jnp.maximum(0, (nch - widx + NW - 1) // NW).astype(jnp.int32)
        cnt_rows = jnp.zeros((NW, LANES), jnp.int32).at[:, 0].set(cnt)
        return rbuf, cbuf, cnt_rows

    row_h0, col_h0, cnt_h0 = _partition(0)
    row_h1, col_h1, cnt_h1 = _partition(HALF)

    _deg = _make_deg()
    _sh = _make_scatter_half()

    deg_pair = _deg(col_p).reshape(NC, N_NODES)
    dinv = _dinv_call(deg_pair)

    def edge_phase(y):
        z0 = _sh(y, row_h0, col_h0, cnt_h0)
        z1 = _sh(y, row_h1, col_h1, cnt_h1)
        return jnp.concatenate([z0, z1[:, :TAIL]], axis=1)

    Ws = [W1, W2, W3, W4, W5, W6, W7, W8, W9]
    bs = [b1, b2, b3, b4, b5, b6, b7, b8, b9]
    w10p = jnp.pad(W10, ((0, 0), (0, F_LAST - W10.shape[1])))
    b10p = jnp.pad(b10, (0, F_LAST - b10.shape[0]))

    y = _mm1_call(x, Ws[0], dinv)
    for i in range(1, 9):
        z = edge_phase(y)
        y = _mid_call(z, y, dinv, bs[i - 1], Ws[i])
    z = edge_phase(y)
    u = _pre10_call(z, y, dinv, bs[8])
    z = edge_phase(u)
    out16 = _fin_call(z, u, dinv, w10p, b10p)
    return out16[:, : b10.shape[0]]


# speedup vs baseline: 2.8512x; 1.5764x over previous
import functools

import jax
import jax.numpy as jnp
from jax import lax
from jax.experimental import pallas as pl
from jax.experimental.pallas import tpu as pltpu
from jax.experimental.pallas import tpu_sc as plsc

N_NODES = 10000
N_EDGES = 320000
HIDDEN = 128
F_LAST = 16

NC = 2
NS = 16
LANES = 16

CHUNK = 128
N_CHUNKS = 2560
PAD_EDGES = N_CHUNKS * CHUNK
SUB_SEG = N_CHUNKS // NS
HIST_ROWS = N_NODES + LANES

HALF = 5120
ACC_ROWS = HALF + LANES


@functools.cache
def _mesh():
    return plsc.VectorSubcoreMesh(
        core_axis_name="c", subcore_axis_name="s", num_cores=NC, num_subcores=NS
    )


@functools.cache
def _make_deg():
    tile_chunks = N_CHUNKS // (NC * NS)

    @functools.partial(
        pl.kernel,
        out_type=jax.ShapeDtypeStruct((NC, 1, N_NODES), jnp.float32),
        mesh=_mesh(),
        scratch_types=[
            pltpu.VMEM((tile_chunks, CHUNK), jnp.int32),
            pltpu.VMEM((CHUNK,), jnp.float32),
            pltpu.VMEM((10240,), jnp.float32),
            pltpu.VMEM_SHARED((HIST_ROWS,), jnp.float32),
        ],
    )
    def _deg_kernel(col2_hbm, deg_out, col_all, ones_v, zbuf, deg_acc):
        c = lax.axis_index("c")
        s = lax.axis_index("s")

        def _fill_ones(i, carry):
            ones_v[pl.ds(i * 16, 16)] = jnp.ones((16,), jnp.float32)
            return carry

        lax.fori_loop(0, CHUNK // 16, _fill_ones, 0)

        def _fill_zero(i, carry):
            zbuf[pl.ds(i * 16, 16)] = jnp.zeros((16,), jnp.float32)
            return carry

        lax.fori_loop(0, 10240 // 16, _fill_zero, 0)

        @pl.when(s == 0)
        def _():
            pltpu.sync_copy(zbuf.at[pl.ds(0, HIST_ROWS)], deg_acc)

        plsc.subcore_barrier()

        base = (c * NS + s) * tile_chunks
        pltpu.sync_copy(col2_hbm.at[pl.ds(base, tile_chunks)], col_all)

        def _step(k, carry):
            pltpu.sync_copy(ones_v, deg_acc.at[col_all.at[k]], add=True)
            return carry

        lax.fori_loop(0, tile_chunks, _step, 0)
        plsc.subcore_barrier()

        @pl.when(s == 0)
        def _():
            pltpu.sync_copy(deg_acc.at[pl.ds(0, N_NODES)], zbuf.at[pl.ds(0, N_NODES)])
            pltpu.sync_copy(zbuf.at[pl.ds(0, N_NODES)], deg_out.at[c, 0])

    return _deg_kernel


@functools.cache
def _make_scatter():
    @functools.partial(
        pl.kernel,
        out_type=jax.ShapeDtypeStruct((NC, HALF, HIDDEN), jnp.float32),
        mesh=_mesh(),
        scratch_types=[
            pltpu.VMEM((SUB_SEG, CHUNK), jnp.int32),
            pltpu.VMEM((SUB_SEG, CHUNK), jnp.int32),
            pltpu.VMEM((CHUNK, HIDDEN), jnp.float32),
            pltpu.VMEM((128, HIDDEN), jnp.float32),
            pltpu.VMEM_SHARED((ACC_ROWS, HIDDEN), jnp.float32),
            pltpu.VMEM((LANES,), jnp.int32),
            pltpu.SemaphoreType.DMA,
        ],
    )
    def _scatter(y_hbm, row2_hbm, col2_hbm, cnt_hbm, z_out,
                 row_all, col_all, rows_v, zbuf, z_acc, cnt_v, sem):
        c = lax.axis_index("c")
        s = lax.axis_index("s")

        def _zero(i, carry):
            for j in range(HIDDEN // 16):
                zbuf[i, pl.ds(j * 16, 16)] = jnp.zeros((16,), jnp.float32)
            return carry

        lax.fori_loop(0, 128, _zero, 0)

        r0 = s * (ACC_ROWS // NS)
        for off, n in ((0, 128), (128, 128), (256, 65)):
            pltpu.sync_copy(zbuf.at[pl.ds(0, n)], z_acc.at[pl.ds(r0 + off, n)])
        plsc.subcore_barrier()

        pltpu.sync_copy(cnt_hbm.at[c * NS + s], cnt_v)
        cnt = cnt_v[...][0]

        base = c * N_CHUNKS + s * SUB_SEG
        pltpu.sync_copy(row2_hbm.at[pl.ds(base, SUB_SEG)], row_all)
        pltpu.sync_copy(col2_hbm.at[pl.ds(base, SUB_SEG)], col_all)

        def _step(k, carry):
            pltpu.async_copy(y_hbm.at[row_all.at[k]], rows_v, sem).wait()
            pltpu.sync_copy(rows_v, z_acc.at[col_all.at[k]], add=True)
            return carry

        lax.fori_loop(0, cnt, _step, 0)
        plsc.subcore_barrier()

        d0 = s * (HALF // NS)
        pltpu.sync_copy(z_acc.at[pl.ds(d0, HALF // NS)], z_out.at[c, pl.ds(d0, HALF // NS)])

    return _scatter


def _dinv_body(deg_ref, dinv_ref):
    dinv_ref[...] = lax.rsqrt(deg_ref[0] + deg_ref[1] + 1.0)


def _dinv_call(deg_pair):
    return pl.pallas_call(
        _dinv_body,
        out_shape=jax.ShapeDtypeStruct((N_NODES, 1), jnp.float32),
    )(deg_pair.reshape(NC, N_NODES, 1))


def _mm1_body(x_ref, w_ref, dinv_ref, y_ref):
    y_ref[...] = (
        jnp.dot(x_ref[...], w_ref[...], preferred_element_type=jnp.float32)
        * dinv_ref[...]
    )


def _mm1_call(x, w, dinv):
    bm = 400
    kdim = x.shape[1]
    return pl.pallas_call(
        _mm1_body,
        grid=(N_NODES // bm,),
        in_specs=[
            pl.BlockSpec((bm, kdim), lambda i: (i, 0)),
            pl.BlockSpec((kdim, HIDDEN), lambda i: (0, 0)),
            pl.BlockSpec((bm, 1), lambda i: (i, 0)),
        ],
        out_specs=pl.BlockSpec((bm, HIDDEN), lambda i: (i, 0)),
        out_shape=jax.ShapeDtypeStruct((N_NODES, HIDDEN), jnp.float32),
    )(x, w, dinv)


def _mid_body(z_ref, y_ref, dinv_ref, b_ref, w_ref, o_ref):
    t = (z_ref[...] + y_ref[...]) * dinv_ref[...] + b_ref[...]
    t = jnp.maximum(t, 0.0)
    o_ref[...] = (
        jnp.dot(t, w_ref[...], preferred_element_type=jnp.float32) * dinv_ref[...]
    )


def _mid_call(z, y, dinv, b, w):
    bm = 1000
    return pl.pallas_call(
        _mid_body,
        grid=(N_NODES // bm,),
        in_specs=[
            pl.BlockSpec((bm, HIDDEN), lambda i: (i, 0)),
            pl.BlockSpec((bm, HIDDEN), lambda i: (i, 0)),
            pl.BlockSpec((bm, 1), lambda i: (i, 0)),
            pl.BlockSpec((1, HIDDEN), lambda i: (0, 0)),
            pl.BlockSpec((HIDDEN, HIDDEN), lambda i: (0, 0)),
        ],
        out_specs=pl.BlockSpec((bm, HIDDEN), lambda i: (i, 0)),
        out_shape=jax.ShapeDtypeStruct((N_NODES, HIDDEN), jnp.float32),
    )(z, y, dinv, b.reshape(1, HIDDEN), w)


def _pre10_body(z_ref, y_ref, dinv_ref, b_ref, o_ref):
    t = (z_ref[...] + y_ref[...]) * dinv_ref[...] + b_ref[...]
    o_ref[...] = jnp.maximum(t, 0.0) * dinv_ref[...]


def _pre10_call(z, y, dinv, b):
    bm = 1000
    return pl.pallas_call(
        _pre10_body,
        grid=(N_NODES // bm,),
        in_specs=[
            pl.BlockSpec((bm, HIDDEN), lambda i: (i, 0)),
            pl.BlockSpec((bm, HIDDEN), lambda i: (i, 0)),
            pl.BlockSpec((bm, 1), lambda i: (i, 0)),
            pl.BlockSpec((1, HIDDEN), lambda i: (0, 0)),
        ],
        out_specs=pl.BlockSpec((bm, HIDDEN), lambda i: (i, 0)),
        out_shape=jax.ShapeDtypeStruct((N_NODES, HIDDEN), jnp.float32),
    )(z, y, dinv, b.reshape(1, HIDDEN))


def _fin_body(z_ref, u_ref, dinv_ref, w_ref, b_ref, o_ref):
    t = (z_ref[...] + u_ref[...]) * dinv_ref[...]
    o_ref[...] = (
        jnp.dot(t, w_ref[...], preferred_element_type=jnp.float32) + b_ref[...]
    )


def _fin_call(z, u, dinv, w, b):
    bm = 1000
    return pl.pallas_call(
        _fin_body,
        grid=(N_NODES // bm,),
        in_specs=[
            pl.BlockSpec((bm, HIDDEN), lambda i: (i, 0)),
            pl.BlockSpec((bm, HIDDEN), lambda i: (i, 0)),
            pl.BlockSpec((bm, 1), lambda i: (i, 0)),
            pl.BlockSpec((HIDDEN, F_LAST), lambda i: (0, 0)),
            pl.BlockSpec((1, F_LAST), lambda i: (0, 0)),
        ],
        out_specs=pl.BlockSpec((bm, F_LAST), lambda i: (i, 0)),
        out_shape=jax.ShapeDtypeStruct((N_NODES, F_LAST), jnp.float32),
    )(z, u, dinv, w, b.reshape(1, F_LAST))


def kernel(x, edge_index, W1, b1, W2, b2, W3, b3, W4, b4, W5, b5, W6, b6,
           W7, b7, W8, b8, W9, b9, W10, b10):
    ei = edge_index.astype(jnp.int32)
    row, col = ei[0], ei[1]
    pad = PAD_EDGES - N_EDGES
    spread = jnp.arange(pad, dtype=jnp.int32) % LANES
    col_p = jnp.concatenate(
        [col, N_NODES + spread]).reshape(N_CHUNKS, CHUNK)

    sub = jnp.arange(NS, dtype=jnp.int32)
    in0 = col < HALF
    pos = jnp.where(in0, jnp.cumsum(in0) - 1, jnp.cumsum(~in0) - 1)
    ch = pos // CHUNK
    addr = (
        jnp.where(in0, 0, PAD_EDGES)
        + (ch % NS) * (SUB_SEG * CHUNK)
        + (ch // NS) * CHUNK
        + pos % CHUNK
    )
    rbuf = jnp.zeros((NC * PAD_EDGES,), jnp.int32)
    cinit = HALF + jnp.arange(NC * PAD_EDGES, dtype=jnp.int32) % LANES
    row2 = rbuf.at[addr].set(row).reshape(NC * N_CHUNKS, CHUNK)
    col2 = cinit.at[addr].set(col - jnp.where(in0, 0, HALF)).reshape(
        NC * N_CHUNKS, CHUNK)
    n0 = jnp.sum(in0.astype(jnp.int32))
    nch = jnp.stack([(n0 + CHUNK - 1) // CHUNK,
                     (N_EDGES - n0 + CHUNK - 1) // CHUNK])
    cnt = jnp.maximum(0, (nch[:, None] - sub[None, :] + NS - 1) // NS)
    cnt_rows = (
        jnp.zeros((NC * NS, LANES), jnp.int32)
        .at[:, 0].set(cnt.reshape(NC * NS).astype(jnp.int32))
    )

    _deg = _make_deg()
    _sc = _make_scatter()

    deg_pair = _deg(col_p).reshape(NC, N_NODES)
    dinv = _dinv_call(deg_pair)

    def edge_phase(y):
        z = _sc(y, row2, col2, cnt_rows)
        return z.reshape(NC * HALF, HIDDEN)

    Ws = [W1, W2, W3, W4, W5, W6, W7, W8, W9]
    bs = [b1, b2, b3, b4, b5, b6, b7, b8, b9]
    w10p = jnp.pad(W10, ((0, 0), (0, F_LAST - W10.shape[1])))
    b10p = jnp.pad(b10, (0, F_LAST - b10.shape[0]))

    y = _mm1_call(x, Ws[0], dinv)
    for i in range(1, 9):
        z = edge_phase(y)
        y = _mid_call(z, y, dinv, bs[i - 1], Ws[i])
    z = edge_phase(y)
    u = _pre10_call(z, y, dinv, bs[8])
    z = edge_phase(u)
    out16 = _fin_call(z, u, dinv, w10p, b10p)
    return out16[:, : b10.shape[0]]
